# baseline (speedup 1.0000x reference)
import functools

import jax
import jax.numpy as jnp
from jax.experimental import pallas as pl

_N = 50000
_H = 128
_BLK = 1000
_EL_BLK = 2000


def _proj_body(x_ref, w_ref, b_ref, o_ref):
    o_ref[...] = (
        jnp.dot(x_ref[...], w_ref[...].T, preferred_element_type=jnp.float32)
        + b_ref[...]
    )


def _proj(x, W, b):
    n = x.shape[0]
    return pl.pallas_call(
        _proj_body,
        grid=(n // _BLK,),
        in_specs=[
            pl.BlockSpec((_BLK, _H), lambda i: (i, 0)),
            pl.BlockSpec((_H, _H), lambda i: (0, 0)),
            pl.BlockSpec((1, _H), lambda i: (0, 0)),
        ],
        out_specs=pl.BlockSpec((_BLK, _H), lambda i: (i, 0)),
        out_shape=jax.ShapeDtypeStruct((n, _H), jnp.float32),
    )(x, W, b.reshape(1, _H))


def _conv_body(norm, act, s_ref, cnt_ref, xd_ref, wl_ref, b_ref, wr_ref, o_ref):
    cnt = jnp.maximum(cnt_ref[...], 1.0)
    mean = s_ref[...] / cnt
    out = (
        jnp.dot(mean, wl_ref[...].T, preferred_element_type=jnp.float32)
        + b_ref[...]
        + jnp.dot(xd_ref[...], wr_ref[...].T, preferred_element_type=jnp.float32)
    )
    if norm:
        nrm = jnp.maximum(jnp.sqrt(jnp.sum(out * out, -1, keepdims=True)), 1e-12)
        out = out / nrm
    if act:
        out = jnp.where(out >= 0, out, 0.1 * out)
    o_ref[...] = out


def _conv(s, cnt, x_dst, Wl, b, Wr, norm, act):
    n = s.shape[0]
    return pl.pallas_call(
        functools.partial(_conv_body, norm, act),
        grid=(n // _BLK,),
        in_specs=[
            pl.BlockSpec((_BLK, _H), lambda i: (i, 0)),
            pl.BlockSpec((_BLK, 1), lambda i: (i, 0)),
            pl.BlockSpec((_BLK, _H), lambda i: (i, 0)),
            pl.BlockSpec((_H, _H), lambda i: (0, 0)),
            pl.BlockSpec((1, _H), lambda i: (0, 0)),
            pl.BlockSpec((_H, _H), lambda i: (0, 0)),
        ],
        out_specs=pl.BlockSpec((_BLK, _H), lambda i: (i, 0)),
        out_shape=jax.ShapeDtypeStruct((n, _H), jnp.float32),
    )(s, cnt.reshape(n, 1), x_dst, Wl, b.reshape(1, _H), Wr)


def _dot_body(a_ref, b_ref, o_ref):
    o_ref[...] = jnp.sum(a_ref[...] * b_ref[...], axis=-1, keepdims=True)


def _edge_dot(te, pe):
    n = te.shape[0]
    out = pl.pallas_call(
        _dot_body,
        grid=(n // _EL_BLK,),
        in_specs=[
            pl.BlockSpec((_EL_BLK, _H), lambda i: (i, 0)),
            pl.BlockSpec((_EL_BLK, _H), lambda i: (i, 0)),
        ],
        out_specs=pl.BlockSpec((_EL_BLK, 1), lambda i: (i, 0)),
        out_shape=jax.ShapeDtypeStruct((n, 1), jnp.float32),
    )(te, pe)
    return out.reshape(n)


def _segsum(x_src, src, dst, n_dst):
    msg = jnp.take(x_src, src, axis=0)
    return jax.ops.segment_sum(msg, dst, num_segments=n_dst)


def kernel(x_track, x_playlist, edge_index_tp, edge_index_pt, edge_label_index,
           Wt, bt, Wp, bp,
           W1_tp_l, W1_tp_r, b1_tp, W1_pt_l, W1_pt_r, b1_pt,
           W2_tp_l, W2_tp_r, b2_tp, W2_pt_l, W2_pt_r, b2_pt,
           W3_tp_l, W3_tp_r, b3_tp, W3_pt_l, W3_pt_r, b3_pt):
    n = x_track.shape[0]
    src_tp, dst_tp = edge_index_tp[0], edge_index_tp[1]
    src_pt, dst_pt = edge_index_pt[0], edge_index_pt[1]
    ones = jnp.ones((src_tp.shape[0],), jnp.float32)
    cnt_tp = jax.ops.segment_sum(ones, dst_tp, num_segments=n)
    cnt_pt = jax.ops.segment_sum(ones, dst_pt, num_segments=n)

    x_t = _proj(x_track, Wt, bt)
    x_p = _proj(x_playlist, Wp, bp)

    params = {
        (1, 'tp'): (W1_tp_l, b1_tp, W1_tp_r), (1, 'pt'): (W1_pt_l, b1_pt, W1_pt_r),
        (2, 'tp'): (W2_tp_l, b2_tp, W2_tp_r), (2, 'pt'): (W2_pt_l, b2_pt, W2_pt_r),
        (3, 'tp'): (W3_tp_l, b3_tp, W3_tp_r), (3, 'pt'): (W3_pt_l, b3_pt, W3_pt_r),
    }
    for l, norm in ((1, True), (2, True), (3, False)):
        Wl_tp, b_tp, Wr_tp = params[(l, 'tp')]
        Wl_pt, b_pt, Wr_pt = params[(l, 'pt')]
        s_p = _segsum(x_t, src_tp, dst_tp, n)
        s_t = _segsum(x_p, src_pt, dst_pt, n)
        act = l < 3
        new_p = _conv(s_p, cnt_tp, x_p, Wl_tp, b_tp, Wr_tp, norm, act)
        new_t = _conv(s_t, cnt_pt, x_t, Wl_pt, b_pt, Wr_pt, norm, act)
        x_t, x_p = new_t, new_p

    te = jnp.take(x_t, edge_label_index[0], axis=0)
    pe = jnp.take(x_p, edge_label_index[1], axis=0)
    return _edge_dot(te, pe)


# speedup vs baseline: 1.9029x; 1.9029x over previous
import functools

import jax
import jax.numpy as jnp
from jax import lax
from jax.experimental import pallas as pl
from jax.experimental.pallas import tpu as pltpu
from jax.experimental.pallas import tpu_sc as plsc

_N = 50000
_H = 128
_BLK = 1000

_E = 500000
_E_PAD = 524288
_ES = 32768
_NB = _ES // 128
_NPAD = 50176
_SHR = _NPAD // 16
_G = 8
_K = 128

_EL = 100000
_EL_PAD = 102400
_LS = 3200
_EL_BLK = 2048

_MESH = plsc.VectorSubcoreMesh(core_axis_name="c", subcore_axis_name="s")


def _segsum_body(x8_hbm, src2_hbm, dst2_hbm, s_out, cnt_out,
                 src2d, dst2d, src_b, rows_v, ones_v, zbuf, cntb, acc, cnt,
                 sem):
    cid = lax.axis_index("c")
    sid = lax.axis_index("s")
    zeros16 = jnp.zeros((16,), jnp.float32)
    for j in range(8):
        ones_v[pl.ds(j * 16, 16)] = jnp.ones((16,), jnp.float32)

    def _zb(i, _):
        zbuf[i, pl.ds(0, 16)] = zeros16
        return 0
    lax.fori_loop(0, zbuf.shape[0], _zb, 0)

    def _zc(i, _):
        cntb[pl.ds(i * 16, 16)] = zeros16
        return 0
    lax.fori_loop(0, _SHR // 16, _zc, 0)

    pltpu.sync_copy(src2_hbm.at[pl.ds(sid * _NB, _NB)], src2d)
    pltpu.sync_copy(dst2_hbm.at[pl.ds(sid * _NB, _NB)], dst2d)

    def _scale(r, _):
        for j in range(8):
            src2d[r, pl.ds(j * 16, 16)] = src2d[r, pl.ds(j * 16, 16)] * 8
        return 0
    lax.fori_loop(0, _NB, _scale, 0)

    for k in range(_G // 2):
        g = cid + 2 * k
        nz = zbuf.shape[0]
        for t in range(_SHR // 392):
            pltpu.sync_copy(zbuf, acc.at[pl.ds(sid * _SHR + t * 392, 392)])
        if k == 0:
            pltpu.sync_copy(cntb, cnt.at[pl.ds(sid * _SHR, _SHR)])
        plsc.subcore_barrier()

        def batch_body(b, _):
            for j in range(8):
                src_b[pl.ds(j * 16, 16)] = src2d[b, pl.ds(j * 16, 16)] + g
            pltpu.async_copy(x8_hbm.at[src_b], rows_v, sem).wait()
            pltpu.sync_copy(rows_v, acc.at[dst2d.at[b]], add=True)
            if k == 0:
                pltpu.sync_copy(ones_v, cnt.at[dst2d.at[b]], add=True)
            return 0

        lax.fori_loop(0, _NB, batch_body, 0)
        plsc.subcore_barrier()

        pltpu.sync_copy(acc.at[pl.ds(sid * _SHR, _SHR)],
                        s_out.at[pl.ds(sid * _SHR, _SHR), pl.ds(g * 16, 16)])
        if k == 0:
            pltpu.sync_copy(cnt.at[pl.ds(sid * _SHR, _SHR)], cntb)
            pltpu.sync_copy(cntb, cnt_out.at[pl.ds(sid * _SHR, _SHR)])
        plsc.subcore_barrier()


def _make_segsum():
    return pl.kernel(
        _segsum_body,
        out_type=[
            jax.ShapeDtypeStruct((_NPAD, _H), jnp.float32),
            jax.ShapeDtypeStruct((_NPAD,), jnp.float32),
        ],
        mesh=_MESH,
        scratch_types=[
            pltpu.VMEM((_NB, _K), jnp.int32),
            pltpu.VMEM((_NB, _K), jnp.int32),
            pltpu.VMEM((_K,), jnp.int32),
            pltpu.VMEM((_K, 16), jnp.float32),
            pltpu.VMEM((_K,), jnp.float32),
            pltpu.VMEM((392, 16), jnp.float32),
            pltpu.VMEM((_SHR,), jnp.float32),
            pltpu.VMEM_SHARED((_NPAD, 16), jnp.float32),
            pltpu.VMEM_SHARED((_NPAD,), jnp.float32),
            pltpu.SemaphoreType.DMA,
        ],
        compiler_params=pltpu.CompilerParams(
            needs_layout_passes=False, use_tc_tiling_on_sc=False),
    )


_segsum_call = _make_segsum()


def _link_gather_body(xt_hbm, xp_hbm, ti_hbm, pi_hbm, te_out, pe_out,
                      tib, pib, rows_v, sem):
    wid = lax.axis_index("c") * 16 + lax.axis_index("s")

    def body(b, _):
        base = wid * _LS + b * _K
        pltpu.sync_copy(ti_hbm.at[pl.ds(base, _K)], tib)
        pltpu.sync_copy(pi_hbm.at[pl.ds(base, _K)], pib)
        pltpu.async_copy(xt_hbm.at[tib], rows_v, sem).wait()
        pltpu.sync_copy(rows_v, te_out.at[pl.ds(base, _K)])
        pltpu.async_copy(xp_hbm.at[pib], rows_v, sem).wait()
        pltpu.sync_copy(rows_v, pe_out.at[pl.ds(base, _K)])
        return 0

    lax.fori_loop(0, _LS // _K, body, 0)


_link_gather = pl.kernel(
    _link_gather_body,
    out_type=[
        jax.ShapeDtypeStruct((_EL_PAD, _H), jnp.float32),
        jax.ShapeDtypeStruct((_EL_PAD, _H), jnp.float32),
    ],
    mesh=_MESH,
    scratch_types=[
        pltpu.VMEM((_K,), jnp.int32),
        pltpu.VMEM((_K,), jnp.int32),
        pltpu.VMEM((_K, _H), jnp.float32),
        pltpu.SemaphoreType.DMA,
    ],
    compiler_params=pltpu.CompilerParams(needs_layout_passes=False),
)


def _proj_body(x_ref, w_ref, b_ref, o_ref):
    o_ref[...] = (
        jnp.dot(x_ref[...], w_ref[...].T, preferred_element_type=jnp.float32)
        + b_ref[...]
    )


def _proj(x, W, b):
    n = x.shape[0]
    return pl.pallas_call(
        _proj_body,
        grid=(n // _BLK,),
        in_specs=[
            pl.BlockSpec((_BLK, _H), lambda i: (i, 0)),
            pl.BlockSpec((_H, _H), lambda i: (0, 0)),
            pl.BlockSpec((1, _H), lambda i: (0, 0)),
        ],
        out_specs=pl.BlockSpec((_BLK, _H), lambda i: (i, 0)),
        out_shape=jax.ShapeDtypeStruct((n, _H), jnp.float32),
    )(x, W, b.reshape(1, _H))


def _conv_body(norm, act, s_ref, cnt_ref, xd_ref, wl_ref, b_ref, wr_ref, o_ref):
    cnt = jnp.maximum(cnt_ref[...], 1.0)
    mean = s_ref[...] / cnt
    out = (
        jnp.dot(mean, wl_ref[...].T, preferred_element_type=jnp.float32)
        + b_ref[...]
        + jnp.dot(xd_ref[...], wr_ref[...].T, preferred_element_type=jnp.float32)
    )
    if norm:
        nrm = jnp.maximum(jnp.sqrt(jnp.sum(out * out, -1, keepdims=True)), 1e-12)
        out = out / nrm
    if act:
        out = jnp.where(out >= 0, out, 0.1 * out)
    o_ref[...] = out


def _conv(s_pad, cnt_pad, x_dst, Wl, b, Wr, norm, act):
    n = x_dst.shape[0]
    return pl.pallas_call(
        functools.partial(_conv_body, norm, act),
        grid=(n // _BLK,),
        in_specs=[
            pl.BlockSpec((_BLK, _H), lambda i: (i, 0)),
            pl.BlockSpec((_BLK, 1), lambda i: (i, 0)),
            pl.BlockSpec((_BLK, _H), lambda i: (i, 0)),
            pl.BlockSpec((_H, _H), lambda i: (0, 0)),
            pl.BlockSpec((1, _H), lambda i: (0, 0)),
            pl.BlockSpec((_H, _H), lambda i: (0, 0)),
        ],
        out_specs=pl.BlockSpec((_BLK, _H), lambda i: (i, 0)),
        out_shape=jax.ShapeDtypeStruct((n, _H), jnp.float32),
    )(s_pad, cnt_pad.reshape(_NPAD, 1), x_dst, Wl, b.reshape(1, _H), Wr)


def _dot_body(a_ref, b_ref, o_ref):
    o_ref[...] = jnp.sum(a_ref[...] * b_ref[...], axis=-1, keepdims=True)


def _edge_dot(te, pe):
    n = te.shape[0]
    out = pl.pallas_call(
        _dot_body,
        grid=(n // _EL_BLK,),
        in_specs=[
            pl.BlockSpec((_EL_BLK, _H), lambda i: (i, 0)),
            pl.BlockSpec((_EL_BLK, _H), lambda i: (i, 0)),
        ],
        out_specs=pl.BlockSpec((_EL_BLK, 1), lambda i: (i, 0)),
        out_shape=jax.ShapeDtypeStruct((n, 1), jnp.float32),
    )(te, pe)
    return out.reshape(n)


def _pad_edges(ei):
    npad = _E_PAD - _E
    src = jnp.concatenate(
        [ei[0].astype(jnp.int32),
         (jnp.arange(npad, dtype=jnp.int32) * 131) % _N])
    dst = jnp.concatenate(
        [ei[1].astype(jnp.int32),
         _N + (jnp.arange(npad, dtype=jnp.int32) % (_NPAD - _N))])
    return src.reshape(_E_PAD // _K, _K), dst.reshape(_E_PAD // _K, _K)


def kernel(x_track, x_playlist, edge_index_tp, edge_index_pt, edge_label_index,
           Wt, bt, Wp, bp,
           W1_tp_l, W1_tp_r, b1_tp, W1_pt_l, W1_pt_r, b1_pt,
           W2_tp_l, W2_tp_r, b2_tp, W2_pt_l, W2_pt_r, b2_pt,
           W3_tp_l, W3_tp_r, b3_tp, W3_pt_l, W3_pt_r, b3_pt):
    src_tp, dst_tp = _pad_edges(edge_index_tp)
    src_pt, dst_pt = _pad_edges(edge_index_pt)

    x_t = _proj(x_track, Wt, bt)
    x_p = _proj(x_playlist, Wp, bp)

    params = {
        (1, 'tp'): (W1_tp_l, b1_tp, W1_tp_r), (1, 'pt'): (W1_pt_l, b1_pt, W1_pt_r),
        (2, 'tp'): (W2_tp_l, b2_tp, W2_tp_r), (2, 'pt'): (W2_pt_l, b2_pt, W2_pt_r),
        (3, 'tp'): (W3_tp_l, b3_tp, W3_tp_r), (3, 'pt'): (W3_pt_l, b3_pt, W3_pt_r),
    }
    cnt_tp = cnt_pt = None
    for l, norm in ((1, True), (2, True), (3, False)):
        Wl_tp, b_tp, Wr_tp = params[(l, 'tp')]
        Wl_pt, b_pt, Wr_pt = params[(l, 'pt')]
        s_p, c_tp = _segsum_call(x_t.reshape(-1, 16), src_tp, dst_tp)
        s_t, c_pt = _segsum_call(x_p.reshape(-1, 16), src_pt, dst_pt)
        if cnt_tp is None:
            cnt_tp, cnt_pt = c_tp, c_pt
        act = l < 3
        new_p = _conv(s_p, cnt_tp, x_p, Wl_tp, b_tp, Wr_tp, norm, act)
        new_t = _conv(s_t, cnt_pt, x_t, Wl_pt, b_pt, Wr_pt, norm, act)
        x_t, x_p = new_t, new_p

    npadl = _EL_PAD - _EL
    ti = jnp.concatenate(
        [edge_label_index[0].astype(jnp.int32),
         (jnp.arange(npadl, dtype=jnp.int32) * 131) % _N])
    pi = jnp.concatenate(
        [edge_label_index[1].astype(jnp.int32),
         (jnp.arange(npadl, dtype=jnp.int32) * 157) % _N])
    te, pe = _link_gather(x_t, x_p, ti, pi)
    return _edge_dot(te, pe)[:_EL]


# speedup vs baseline: 2.0761x; 1.0910x over previous
import functools

import jax
import jax.numpy as jnp
from jax import lax
from jax.experimental import pallas as pl
from jax.experimental.pallas import tpu as pltpu
from jax.experimental.pallas import tpu_sc as plsc

_N = 50000
_H = 128
_BLK = 1000

_E = 500000
_E_PAD = 524288
_ES = 32768
_NB = _ES // 128
_NPAD = 50176
_SHR = _NPAD // 16
_G = 16
_CW = _H // _G
_K = 128
_KB = 1024

_EL = 100000
_EL_PAD = 102400
_LS = 3200
_EL_BLK = 2048

_MESH = plsc.VectorSubcoreMesh(core_axis_name="c", subcore_axis_name="s")


def _segsum_body(x8_hbm, src_hbm, dst_hbm, tok_hbm, s_out, cnt_out,
                 src1d, dst1d, src_b, dst_b, rows_v, ones_v, zbuf, cntb,
                 acc, cnt, sem):
    cid = lax.axis_index("c")
    sid = lax.axis_index("s")
    pltpu.sync_copy(tok_hbm.at[pl.ds(0, 16)], cntb.at[pl.ds(0, 16)])
    zeros16 = jnp.zeros((16,), jnp.float32)
    ones16 = jnp.ones((16,), jnp.float32)

    def _zo(i, _):
        ones_v[pl.ds(i * 16, 16)] = ones16
        return 0
    lax.fori_loop(0, _KB // 16, _zo, 0)

    def _zb(i, _):
        zbuf[i, pl.ds(0, 16)] = zeros16
        return 0
    lax.fori_loop(0, zbuf.shape[0], _zb, 0)

    def _zc(i, _):
        cntb[pl.ds(i * 16, 16)] = zeros16
        return 0
    lax.fori_loop(0, _SHR // 16, _zc, 0)

    pltpu.sync_copy(src_hbm.at[pl.ds(sid * _ES, _ES)], src1d)
    pltpu.sync_copy(dst_hbm.at[pl.ds(sid * _ES, _ES)], dst1d)

    def _scale(r, _):
        src1d[pl.ds(r * 16, 16)] = src1d[pl.ds(r * 16, 16)] * _G + cid
        return 0
    lax.fori_loop(0, _ES // 16, _scale, 0)

    for k in range(_G // 2):
        g = cid + 2 * k
        for t in range(_SHR // 392):
            pltpu.sync_copy(zbuf, acc.at[pl.ds(sid * _SHR + t * 392, 392)])
        if k == 0:
            pltpu.sync_copy(cntb, cnt.at[pl.ds(sid * _SHR, _SHR)])
        plsc.subcore_barrier()

        def batch_body(b, _):
            for r in range(_KB // 16):
                src_b[pl.ds(r * 16, 16)] = src1d[pl.ds(b * _KB + r * 16, 16)]
                dst_b[pl.ds(r * 16, 16)] = dst1d[pl.ds(b * _KB + r * 16, 16)]
            pltpu.async_copy(x8_hbm.at[src_b], rows_v, sem).wait()
            pltpu.sync_copy(rows_v, acc.at[dst_b], add=True)
            if k == 0:
                pltpu.sync_copy(ones_v, cnt.at[dst_b], add=True)
            return 0

        lax.fori_loop(0, _ES // _KB, batch_body, 0)
        plsc.subcore_barrier()

        pltpu.sync_copy(acc.at[pl.ds(sid * _SHR, _SHR)],
                        s_out.at[pl.ds(sid * _SHR, _SHR), pl.ds(g * _CW, _CW)])
        if k == 0:
            pltpu.sync_copy(cnt.at[pl.ds(sid * _SHR, _SHR)], cntb)
            pltpu.sync_copy(cntb, cnt_out.at[pl.ds(sid * _SHR, _SHR)])
        plsc.subcore_barrier()

        if k < _G // 2 - 1:
            def _adv(r, _):
                src1d[pl.ds(r * 16, 16)] = src1d[pl.ds(r * 16, 16)] + 2
                return 0
            lax.fori_loop(0, _ES // 16, _adv, 0)


def _make_segsum():
    return pl.kernel(
        _segsum_body,
        out_type=[
            jax.ShapeDtypeStruct((_NPAD, _H), jnp.float32),
            jax.ShapeDtypeStruct((_NPAD,), jnp.float32),
        ],
        mesh=_MESH,
        scratch_types=[
            pltpu.VMEM((_ES,), jnp.int32),
            pltpu.VMEM((_ES,), jnp.int32),
            pltpu.VMEM((_KB,), jnp.int32),
            pltpu.VMEM((_KB,), jnp.int32),
            pltpu.VMEM((_KB, _CW), jnp.float32),
            pltpu.VMEM((_KB,), jnp.float32),
            pltpu.VMEM((392, _CW), jnp.float32),
            pltpu.VMEM((_SHR,), jnp.float32),
            pltpu.VMEM_SHARED((_NPAD, _CW), jnp.float32),
            pltpu.VMEM_SHARED((_NPAD,), jnp.float32),
            pltpu.SemaphoreType.DMA,
        ],
        compiler_params=pltpu.CompilerParams(
            needs_layout_passes=False, use_tc_tiling_on_sc=False),
    )


_segsum_call = _make_segsum()


def _link_gather_body(xt_hbm, xp_hbm, ti_hbm, pi_hbm, te_out, pe_out,
                      tib, pib, rows_v, sem):
    wid = lax.axis_index("c") * 16 + lax.axis_index("s")

    def body(b, _):
        base = wid * _LS + b * _K
        pltpu.sync_copy(ti_hbm.at[pl.ds(base, _K)], tib)
        pltpu.sync_copy(pi_hbm.at[pl.ds(base, _K)], pib)
        pltpu.async_copy(xt_hbm.at[tib], rows_v, sem).wait()
        pltpu.sync_copy(rows_v, te_out.at[pl.ds(base, _K)])
        pltpu.async_copy(xp_hbm.at[pib], rows_v, sem).wait()
        pltpu.sync_copy(rows_v, pe_out.at[pl.ds(base, _K)])
        return 0

    lax.fori_loop(0, _LS // _K, body, 0)


_link_gather = pl.kernel(
    _link_gather_body,
    out_type=[
        jax.ShapeDtypeStruct((_EL_PAD, _H), jnp.float32),
        jax.ShapeDtypeStruct((_EL_PAD, _H), jnp.float32),
    ],
    mesh=_MESH,
    scratch_types=[
        pltpu.VMEM((_K,), jnp.int32),
        pltpu.VMEM((_K,), jnp.int32),
        pltpu.VMEM((_K, _H), jnp.float32),
        pltpu.SemaphoreType.DMA,
    ],
    compiler_params=pltpu.CompilerParams(needs_layout_passes=False),
)


def _proj_body(x_ref, w_ref, b_ref, o_ref):
    o_ref[...] = (
        jnp.dot(x_ref[...], w_ref[...].T, preferred_element_type=jnp.float32)
        + b_ref[...]
    )


def _proj(x, W, b):
    n = x.shape[0]
    return pl.pallas_call(
        _proj_body,
        grid=(n // _BLK,),
        in_specs=[
            pl.BlockSpec((_BLK, _H), lambda i: (i, 0)),
            pl.BlockSpec((_H, _H), lambda i: (0, 0)),
            pl.BlockSpec((1, _H), lambda i: (0, 0)),
        ],
        out_specs=pl.BlockSpec((_BLK, _H), lambda i: (i, 0)),
        out_shape=jax.ShapeDtypeStruct((n, _H), jnp.float32),
    )(x, W, b.reshape(1, _H))


def _conv_body(norm, act, s_ref, cnt_ref, xd_ref, wl_ref, b_ref, wr_ref, o_ref):
    cnt = jnp.maximum(cnt_ref[...], 1.0)
    mean = s_ref[...] / cnt
    out = (
        jnp.dot(mean, wl_ref[...].T, preferred_element_type=jnp.float32)
        + b_ref[...]
        + jnp.dot(xd_ref[...], wr_ref[...].T, preferred_element_type=jnp.float32)
    )
    if norm:
        nrm = jnp.maximum(jnp.sqrt(jnp.sum(out * out, -1, keepdims=True)), 1e-12)
        out = out / nrm
    if act:
        out = jnp.where(out >= 0, out, 0.1 * out)
    o_ref[...] = out


def _conv(s_pad, cnt_pad, x_dst, Wl, b, Wr, norm, act):
    n = x_dst.shape[0]
    return pl.pallas_call(
        functools.partial(_conv_body, norm, act),
        grid=(n // _BLK,),
        in_specs=[
            pl.BlockSpec((_BLK, _H), lambda i: (i, 0)),
            pl.BlockSpec((_BLK, 1), lambda i: (i, 0)),
            pl.BlockSpec((_BLK, _H), lambda i: (i, 0)),
            pl.BlockSpec((_H, _H), lambda i: (0, 0)),
            pl.BlockSpec((1, _H), lambda i: (0, 0)),
            pl.BlockSpec((_H, _H), lambda i: (0, 0)),
        ],
        out_specs=pl.BlockSpec((_BLK, _H), lambda i: (i, 0)),
        out_shape=jax.ShapeDtypeStruct((n, _H), jnp.float32),
    )(s_pad, cnt_pad.reshape(_NPAD, 1), x_dst, Wl, b.reshape(1, _H), Wr)


def _dot_body(a_ref, b_ref, o_ref):
    o_ref[...] = jnp.sum(a_ref[...] * b_ref[...], axis=-1, keepdims=True)


def _edge_dot(te, pe):
    n = te.shape[0]
    out = pl.pallas_call(
        _dot_body,
        grid=(n // _EL_BLK,),
        in_specs=[
            pl.BlockSpec((_EL_BLK, _H), lambda i: (i, 0)),
            pl.BlockSpec((_EL_BLK, _H), lambda i: (i, 0)),
        ],
        out_specs=pl.BlockSpec((_EL_BLK, 1), lambda i: (i, 0)),
        out_shape=jax.ShapeDtypeStruct((n, 1), jnp.float32),
    )(te, pe)
    return out.reshape(n)


def _pad_edges(ei):
    npad = _E_PAD - _E
    src = jnp.concatenate(
        [ei[0].astype(jnp.int32),
         (jnp.arange(npad, dtype=jnp.int32) * 131) % _N])
    dst = jnp.concatenate(
        [ei[1].astype(jnp.int32),
         _N + (jnp.arange(npad, dtype=jnp.int32) % (_NPAD - _N))])
    return src, dst


def kernel(x_track, x_playlist, edge_index_tp, edge_index_pt, edge_label_index,
           Wt, bt, Wp, bp,
           W1_tp_l, W1_tp_r, b1_tp, W1_pt_l, W1_pt_r, b1_pt,
           W2_tp_l, W2_tp_r, b2_tp, W2_pt_l, W2_pt_r, b2_pt,
           W3_tp_l, W3_tp_r, b3_tp, W3_pt_l, W3_pt_r, b3_pt):
    src_tp, dst_tp = _pad_edges(edge_index_tp)
    src_pt, dst_pt = _pad_edges(edge_index_pt)
    tok = jnp.zeros((16,), jnp.float32)

    x_t = _proj(x_track, Wt, bt)
    x_p = _proj(x_playlist, Wp, bp)

    params = {
        (1, 'tp'): (W1_tp_l, b1_tp, W1_tp_r), (1, 'pt'): (W1_pt_l, b1_pt, W1_pt_r),
        (2, 'tp'): (W2_tp_l, b2_tp, W2_tp_r), (2, 'pt'): (W2_pt_l, b2_pt, W2_pt_r),
        (3, 'tp'): (W3_tp_l, b3_tp, W3_tp_r), (3, 'pt'): (W3_pt_l, b3_pt, W3_pt_r),
    }
    cnt_tp = cnt_pt = None
    for l, norm in ((1, True), (2, True), (3, False)):
        Wl_tp, b_tp, Wr_tp = params[(l, 'tp')]
        Wl_pt, b_pt, Wr_pt = params[(l, 'pt')]
        s_p, c_tp = _segsum_call(x_t.reshape(-1, _CW), src_tp, dst_tp, tok)
        s_t, c_pt = _segsum_call(x_p.reshape(-1, _CW), src_pt, dst_pt, c_tp)
        tok = c_pt
        if cnt_tp is None:
            cnt_tp, cnt_pt = c_tp, c_pt
        act = l < 3
        new_p = _conv(s_p, cnt_tp, x_p, Wl_tp, b_tp, Wr_tp, norm, act)
        new_t = _conv(s_t, cnt_pt, x_t, Wl_pt, b_pt, Wr_pt, norm, act)
        x_t, x_p = new_t, new_p

    npadl = _EL_PAD - _EL
    ti = jnp.concatenate(
        [edge_label_index[0].astype(jnp.int32),
         (jnp.arange(npadl, dtype=jnp.int32) * 131) % _N])
    pi = jnp.concatenate(
        [edge_label_index[1].astype(jnp.int32),
         (jnp.arange(npadl, dtype=jnp.int32) * 157) % _N])
    te, pe = _link_gather(x_t, x_p, ti, pi)
    return _edge_dot(te, pe)[:_EL]


# speedup vs baseline: 2.5553x; 1.2308x over previous
import functools

import jax
import jax.numpy as jnp
from jax import lax
from jax.experimental import pallas as pl
from jax.experimental.pallas import tpu as pltpu
from jax.experimental.pallas import tpu_sc as plsc

_N = 50000
_H = 128
_BLK = 1000

_E = 500000
_E_PAD = 524288
_ES = 32768
_NB = _ES // 128
_NPAD = 50176
_SHR = _NPAD // 16
_G = 16
_CW = _H // _G
_K = 128
_KB = 512

_EL = 100000
_EL_PAD = 102400
_LS = 3200
_EL_BLK = 2048

_MESH = plsc.VectorSubcoreMesh(core_axis_name="c", subcore_axis_name="s")


def _segsum_pair_body(xt8_hbm, xp8_hbm, srct_hbm, dstt_hbm, srcp_hbm,
                      dstp_hbm, tok_hbm, sp_out, st_out, cntt_out, cntp_out,
                      src1d, dst1d, dst_ba, dst_bb, rows_a, rows_b, ones_v,
                      zbuf, cntb, acc, cnt, sem_a, sem_b):
    cid = lax.axis_index("c")
    sid = lax.axis_index("s")
    zeros16 = jnp.zeros((16,), jnp.float32)
    ones16 = jnp.ones((16,), jnp.float32)
    pltpu.sync_copy(tok_hbm.at[pl.ds(0, 16)], cntb.at[pl.ds(0, 16)])

    def _zo(i, _):
        ones_v[pl.ds(i * 16, 16)] = ones16
        return 0
    lax.fori_loop(0, _KB // 16, _zo, 0)

    def _zb(i, _):
        for j in range(_CW // 16):
            zbuf[i, pl.ds(j * 16, 16)] = zeros16
        return 0
    lax.fori_loop(0, zbuf.shape[0], _zb, 0)

    nzc = _SHR // zbuf.shape[0]

    for x8_hbm, src_hbm, dst_hbm, s_out, cnt_out in (
            (xt8_hbm, srct_hbm, dstt_hbm, sp_out, cntt_out),
            (xp8_hbm, srcp_hbm, dstp_hbm, st_out, cntp_out)):
        def _zc(i, _):
            cntb[pl.ds(i * 16, 16)] = zeros16
            return 0
        lax.fori_loop(0, _SHR // 16, _zc, 0)

        pltpu.sync_copy(src_hbm.at[pl.ds(sid * _ES, _ES)], src1d)
        pltpu.sync_copy(dst_hbm.at[pl.ds(sid * _ES, _ES)], dst1d)

        def _scale(r, _):
            src1d[pl.ds(r * 16, 16)] = src1d[pl.ds(r * 16, 16)] * _G + cid
            return 0
        lax.fori_loop(0, _ES // 16, _scale, 0)

        for k in range(_G // 2):
            g = cid + 2 * k
            for t in range(nzc):
                pltpu.sync_copy(
                    zbuf, acc.at[pl.ds(sid * _SHR + t * zbuf.shape[0],
                                       zbuf.shape[0])])
            if k == 0:
                pltpu.sync_copy(cntb, cnt.at[pl.ds(sid * _SHR, _SHR)])
            plsc.subcore_barrier()

            def pair_body(h, _):
                b0 = 2 * h
                b1 = 2 * h + 1
                for r in range(_KB // 16):
                    dst_ba[pl.ds(r * 16, 16)] = dst1d[
                        pl.ds(b0 * _KB + r * 16, 16)]
                cpa = pltpu.async_copy(
                    x8_hbm.at[src1d.at[pl.ds(b0 * _KB, _KB)]], rows_a, sem_a)
                for r in range(_KB // 16):
                    dst_bb[pl.ds(r * 16, 16)] = dst1d[
                        pl.ds(b1 * _KB + r * 16, 16)]
                cpb = pltpu.async_copy(
                    x8_hbm.at[src1d.at[pl.ds(b1 * _KB, _KB)]], rows_b, sem_b)
                cpa.wait()
                pltpu.sync_copy(rows_a, acc.at[dst_ba], add=True)
                if k == 0:
                    pltpu.sync_copy(ones_v, cnt.at[dst_ba], add=True)
                cpb.wait()
                pltpu.sync_copy(rows_b, acc.at[dst_bb], add=True)
                if k == 0:
                    pltpu.sync_copy(ones_v, cnt.at[dst_bb], add=True)
                return 0

            lax.fori_loop(0, _ES // _KB // 2, pair_body, 0)
            plsc.subcore_barrier()

            pltpu.sync_copy(
                acc.at[pl.ds(sid * _SHR, _SHR)],
                s_out.at[pl.ds(sid * _SHR, _SHR), pl.ds(g * _CW, _CW)])
            if k == 0:
                pltpu.sync_copy(cnt.at[pl.ds(sid * _SHR, _SHR)], cntb)
                pltpu.sync_copy(cntb, cnt_out.at[pl.ds(sid * _SHR, _SHR)])
            plsc.subcore_barrier()

            if k < _G // 2 - 1:
                def _adv(r, _):
                    src1d[pl.ds(r * 16, 16)] = src1d[pl.ds(r * 16, 16)] + 2
                    return 0
                lax.fori_loop(0, _ES // 16, _adv, 0)


def _make_segsum():
    return pl.kernel(
        _segsum_pair_body,
        out_type=[
            jax.ShapeDtypeStruct((_NPAD, _H), jnp.float32),
            jax.ShapeDtypeStruct((_NPAD, _H), jnp.float32),
            jax.ShapeDtypeStruct((_NPAD,), jnp.float32),
            jax.ShapeDtypeStruct((_NPAD,), jnp.float32),
        ],
        mesh=_MESH,
        scratch_types=[
            pltpu.VMEM((_ES,), jnp.int32),
            pltpu.VMEM((_ES,), jnp.int32),
            pltpu.VMEM((_KB,), jnp.int32),
            pltpu.VMEM((_KB,), jnp.int32),
            pltpu.VMEM((_KB, _CW), jnp.float32),
            pltpu.VMEM((_KB, _CW), jnp.float32),
            pltpu.VMEM((_KB,), jnp.float32),
            pltpu.VMEM((196, _CW), jnp.float32),
            pltpu.VMEM((_SHR,), jnp.float32),
            pltpu.VMEM_SHARED((_NPAD, _CW), jnp.float32),
            pltpu.VMEM_SHARED((_NPAD,), jnp.float32),
            pltpu.SemaphoreType.DMA,
            pltpu.SemaphoreType.DMA,
        ],
        compiler_params=pltpu.CompilerParams(
            needs_layout_passes=False, use_tc_tiling_on_sc=False),
    )


_segsum_call = _make_segsum()


def _link_gather_body(xt_hbm, xp_hbm, ti_hbm, pi_hbm, te_out, pe_out,
                      tib, pib, rows_v, sem):
    wid = lax.axis_index("c") * 16 + lax.axis_index("s")

    def body(b, _):
        base = wid * _LS + b * _K
        pltpu.sync_copy(ti_hbm.at[pl.ds(base, _K)], tib)
        pltpu.sync_copy(pi_hbm.at[pl.ds(base, _K)], pib)
        pltpu.async_copy(xt_hbm.at[tib], rows_v, sem).wait()
        pltpu.sync_copy(rows_v, te_out.at[pl.ds(base, _K)])
        pltpu.async_copy(xp_hbm.at[pib], rows_v, sem).wait()
        pltpu.sync_copy(rows_v, pe_out.at[pl.ds(base, _K)])
        return 0

    lax.fori_loop(0, _LS // _K, body, 0)


_link_gather = pl.kernel(
    _link_gather_body,
    out_type=[
        jax.ShapeDtypeStruct((_EL_PAD, _H), jnp.float32),
        jax.ShapeDtypeStruct((_EL_PAD, _H), jnp.float32),
    ],
    mesh=_MESH,
    scratch_types=[
        pltpu.VMEM((_K,), jnp.int32),
        pltpu.VMEM((_K,), jnp.int32),
        pltpu.VMEM((_K, _H), jnp.float32),
        pltpu.SemaphoreType.DMA,
    ],
    compiler_params=pltpu.CompilerParams(needs_layout_passes=False),
)


def _proj_body(x_ref, w_ref, b_ref, o_ref):
    o_ref[...] = (
        jnp.dot(x_ref[...], w_ref[...].T, preferred_element_type=jnp.float32)
        + b_ref[...]
    )


def _proj(x, W, b):
    n = x.shape[0]
    return pl.pallas_call(
        _proj_body,
        grid=(n // _BLK,),
        in_specs=[
            pl.BlockSpec((_BLK, _H), lambda i: (i, 0)),
            pl.BlockSpec((_H, _H), lambda i: (0, 0)),
            pl.BlockSpec((1, _H), lambda i: (0, 0)),
        ],
        out_specs=pl.BlockSpec((_BLK, _H), lambda i: (i, 0)),
        out_shape=jax.ShapeDtypeStruct((n, _H), jnp.float32),
    )(x, W, b.reshape(1, _H))


def _conv_body(norm, act, s_ref, cnt_ref, xd_ref, wl_ref, b_ref, wr_ref, o_ref):
    cnt = jnp.maximum(cnt_ref[...], 1.0)
    mean = s_ref[...] / cnt
    out = (
        jnp.dot(mean, wl_ref[...].T, preferred_element_type=jnp.float32)
        + b_ref[...]
        + jnp.dot(xd_ref[...], wr_ref[...].T, preferred_element_type=jnp.float32)
    )
    if norm:
        nrm = jnp.maximum(jnp.sqrt(jnp.sum(out * out, -1, keepdims=True)), 1e-12)
        out = out / nrm
    if act:
        out = jnp.where(out >= 0, out, 0.1 * out)
    o_ref[...] = out


def _conv(s_pad, cnt_pad, x_dst, Wl, b, Wr, norm, act):
    n = x_dst.shape[0]
    return pl.pallas_call(
        functools.partial(_conv_body, norm, act),
        grid=(n // _BLK,),
        in_specs=[
            pl.BlockSpec((_BLK, _H), lambda i: (i, 0)),
            pl.BlockSpec((_BLK, 1), lambda i: (i, 0)),
            pl.BlockSpec((_BLK, _H), lambda i: (i, 0)),
            pl.BlockSpec((_H, _H), lambda i: (0, 0)),
            pl.BlockSpec((1, _H), lambda i: (0, 0)),
            pl.BlockSpec((_H, _H), lambda i: (0, 0)),
        ],
        out_specs=pl.BlockSpec((_BLK, _H), lambda i: (i, 0)),
        out_shape=jax.ShapeDtypeStruct((n, _H), jnp.float32),
    )(s_pad, cnt_pad.reshape(_NPAD, 1), x_dst, Wl, b.reshape(1, _H), Wr)


def _dot_body(a_ref, b_ref, o_ref):
    o_ref[...] = jnp.sum(a_ref[...] * b_ref[...], axis=-1, keepdims=True)


def _edge_dot(te, pe):
    n = te.shape[0]
    out = pl.pallas_call(
        _dot_body,
        grid=(n // _EL_BLK,),
        in_specs=[
            pl.BlockSpec((_EL_BLK, _H), lambda i: (i, 0)),
            pl.BlockSpec((_EL_BLK, _H), lambda i: (i, 0)),
        ],
        out_specs=pl.BlockSpec((_EL_BLK, 1), lambda i: (i, 0)),
        out_shape=jax.ShapeDtypeStruct((n, 1), jnp.float32),
    )(te, pe)
    return out.reshape(n)


def _pad_edges(ei):
    npad = _E_PAD - _E
    src = jnp.concatenate(
        [ei[0].astype(jnp.int32),
         (jnp.arange(npad, dtype=jnp.int32) * 131) % _N])
    dst = jnp.concatenate(
        [ei[1].astype(jnp.int32),
         _N + (jnp.arange(npad, dtype=jnp.int32) % (_NPAD - _N))])
    return src, dst


def kernel(x_track, x_playlist, edge_index_tp, edge_index_pt, edge_label_index,
           Wt, bt, Wp, bp,
           W1_tp_l, W1_tp_r, b1_tp, W1_pt_l, W1_pt_r, b1_pt,
           W2_tp_l, W2_tp_r, b2_tp, W2_pt_l, W2_pt_r, b2_pt,
           W3_tp_l, W3_tp_r, b3_tp, W3_pt_l, W3_pt_r, b3_pt):
    src_tp, dst_tp = _pad_edges(edge_index_tp)
    src_pt, dst_pt = _pad_edges(edge_index_pt)
    tok = jnp.zeros((16,), jnp.float32)

    x_t = _proj(x_track, Wt, bt)
    x_p = _proj(x_playlist, Wp, bp)

    params = {
        (1, 'tp'): (W1_tp_l, b1_tp, W1_tp_r), (1, 'pt'): (W1_pt_l, b1_pt, W1_pt_r),
        (2, 'tp'): (W2_tp_l, b2_tp, W2_tp_r), (2, 'pt'): (W2_pt_l, b2_pt, W2_pt_r),
        (3, 'tp'): (W3_tp_l, b3_tp, W3_tp_r), (3, 'pt'): (W3_pt_l, b3_pt, W3_pt_r),
    }
    cnt_tp = cnt_pt = None
    for l, norm in ((1, True), (2, True), (3, False)):
        Wl_tp, b_tp, Wr_tp = params[(l, 'tp')]
        Wl_pt, b_pt, Wr_pt = params[(l, 'pt')]
        s_p, s_t, c_tp, c_pt = _segsum_call(
            x_t.reshape(-1, _CW), x_p.reshape(-1, _CW),
            src_tp, dst_tp, src_pt, dst_pt, tok)
        tok = c_pt
        if cnt_tp is None:
            cnt_tp, cnt_pt = c_tp, c_pt
        act = l < 3
        new_p = _conv(s_p, cnt_tp, x_p, Wl_tp, b_tp, Wr_tp, norm, act)
        new_t = _conv(s_t, cnt_pt, x_t, Wl_pt, b_pt, Wr_pt, norm, act)
        x_t, x_p = new_t, new_p

    npadl = _EL_PAD - _EL
    ti = jnp.concatenate(
        [edge_label_index[0].astype(jnp.int32),
         (jnp.arange(npadl, dtype=jnp.int32) * 131) % _N])
    pi = jnp.concatenate(
        [edge_label_index[1].astype(jnp.int32),
         (jnp.arange(npadl, dtype=jnp.int32) * 157) % _N])
    te, pe = _link_gather(x_t, x_p, ti, pi)
    return _edge_dot(te, pe)[:_EL]
